# baseline (speedup 1.0000x reference)
import functools

import jax
import jax.numpy as jnp
from jax import lax
from jax.experimental import pallas as pl
from jax.experimental.pallas import tpu as pltpu, tpu_sc as plsc

VOCAB = 1000000
DIM = 64
ROWS, COLS = 4096, 200
B = ROWS * COLS

NC, NS = 2, 16
NW = NC * NS
SEG = 128
G = 4
GROUP = SEG * G
SEGS_PER_W = B // (NW * SEG)
NG = SEGS_PER_W // G


def _body(x_hbm, table_hbm, out_hbm, idx_all, rows0, rows1, gsem0, gsem1):
    wid = lax.axis_index("s") * NC + lax.axis_index("c")
    seg0 = wid * SEGS_PER_W

    pltpu.sync_copy(x_hbm.at[pl.ds(seg0, SEGS_PER_W)], idx_all)

    rows = (rows0, rows1)
    gsem = (gsem0, gsem1)

    def fire(g, buf):
        for j in range(G):
            pltpu.async_copy(
                table_hbm.at[idx_all.at[g * G + j]],
                rows[buf].at[pl.ds(j * SEG, SEG)],
                gsem[buf],
            )

    def drain(g, buf):
        for j in range(G):
            pltpu.make_async_copy(
                table_hbm.at[idx_all.at[g * G + j]],
                rows[buf].at[pl.ds(j * SEG, SEG)],
                gsem[buf],
            ).wait()

    fire(0, 0)

    @pl.loop(0, NG, step=2)
    def _(g0):
        for b in range(2):
            g = g0 + b
            drain(g, b)

            @pl.when(g + 1 < NG)
            def _():
                fire(g + 1, 1 - b)

            out0 = (seg0 * SEG) + g * GROUP
            pltpu.sync_copy(rows[b], out_hbm.at[pl.ds(out0, GROUP)])


@functools.partial(jax.jit, static_argnames=())
def _lookup(x_flat, table):
    mesh = plsc.VectorSubcoreMesh(core_axis_name="c", subcore_axis_name="s")
    k = pl.kernel(
        _body,
        out_type=jax.ShapeDtypeStruct((B, DIM), jnp.float32),
        mesh=mesh,
        scratch_types=[
            pltpu.VMEM((SEGS_PER_W, SEG), jnp.int32),
            pltpu.VMEM((GROUP, DIM), jnp.float32),
            pltpu.VMEM((GROUP, DIM), jnp.float32),
            pltpu.SemaphoreType.DMA,
            pltpu.SemaphoreType.DMA,
        ],
        compiler_params=pltpu.CompilerParams(use_tc_tiling_on_sc=False),
    )
    return k(x_flat, table)


def kernel(x, table):
    x_flat = jnp.reshape(x.astype(jnp.int32), (B // SEG, SEG))
    out = _lookup(x_flat, table)
    return jnp.reshape(out, (ROWS, COLS, DIM))
